# Initial kernel scaffold; baseline (speedup 1.0000x reference)
import jax
import jax.numpy as jnp
from jax.experimental import pallas as pl


def kernel(points_ref, points):
    raise NotImplementedError("write your pallas kernel here")


# speedup vs baseline: 9.1390x; 9.1390x over previous
import functools

import jax
import jax.numpy as jnp
from jax.experimental import pallas as pl
from jax.experimental.pallas import tpu as pltpu

_K = 16
_ROWS = 256


def _tile_body(pref_full, ppts_full, pref_tile, ppts_tile, out_ref, *, n):
    j = pl.program_id(1)
    a = pref_full[0]
    p = ppts_full[0]
    at = pref_tile[0]
    pt = ppts_tile[0]

    sq_all = jnp.sum(a * a, axis=0, keepdims=True)
    sqp_all = jnp.sum(p * p, axis=0, keepdims=True)
    ones = jnp.ones((8, 1), jnp.float32)
    dd = (((0,), (0,)), ((), ()))
    sq_t = jax.lax.dot_general(at * at, ones, dd)
    sqp_t = jax.lax.dot_general(pt * pt, ones, dd)
    r = jax.lax.dot_general(at, a, dd)
    rp = jax.lax.dot_general(pt, p, dd)
    d2 = sq_t + sq_all - 2.0 * r
    d2p = sqp_t + sqp_all - 2.0 * rp

    cols = jax.lax.broadcasted_iota(jnp.int32, (_ROWS, n), 1)
    rows_g = jax.lax.broadcasted_iota(jnp.int32, (_ROWS, n), 0) + j * _ROWS
    d2 = jnp.where(cols == rows_g, jnp.inf, d2)

    def body(_, carry):
        d2, acc = carry
        m = jnp.min(d2, axis=1, keepdims=True)
        amin = jnp.min(jnp.where(d2 == m, cols, jnp.int32(n)),
                       axis=1, keepdims=True)
        onehot = cols == amin
        dp = jnp.sum(jnp.where(onehot, d2p, 0.0), axis=1, keepdims=True)
        d2 = jnp.where(onehot, jnp.inf, d2)
        s = jnp.maximum(jnp.sqrt(dp / m) - 1.0, 0.0)
        return d2, acc + jnp.sum(s)

    _, total = jax.lax.fori_loop(0, _K, body, (d2, jnp.float32(0.0)))
    out_ref[0, 0, 0, 0] = total


def kernel(points_ref, points):
    b, n, _ = points_ref.shape
    nb = n // _ROWS
    pref = jnp.pad(jnp.transpose(points_ref, (0, 2, 1)),
                   ((0, 0), (0, 5), (0, 0)))
    ppts = jnp.pad(jnp.transpose(points, (0, 2, 1)),
                   ((0, 0), (0, 5), (0, 0)))

    full_spec = pl.BlockSpec((1, 8, n), lambda bi, ji: (bi, 0, 0))
    tile_spec = pl.BlockSpec((1, 8, _ROWS), lambda bi, ji: (bi, 0, ji))
    out = pl.pallas_call(
        functools.partial(_tile_body, n=n),
        grid=(b, nb),
        in_specs=[full_spec, full_spec, tile_spec, tile_spec],
        out_specs=pl.BlockSpec((1, 1, 1, 1), lambda bi, ji: (bi, ji, 0, 0),
                               memory_space=pltpu.SMEM),
        out_shape=jax.ShapeDtypeStruct((b, nb, 1, 1), jnp.float32),
    )(pref, ppts, pref, ppts)
    return jnp.sum(out) / jnp.float32(b * n * _K)


# speedup vs baseline: 14.7768x; 1.6169x over previous
import functools

import jax
import jax.numpy as jnp
from jax import lax
from jax.experimental import pallas as pl
from jax.experimental.pallas import tpu as pltpu
from jax.experimental.pallas import tpu_sc as plsc

_N = 4096
_K = 16
_L = 16
_BLKW = 128
_NBLK = _N // _BLKW
_NW = 32


def _splat(ref, gi):
    return plsc.load_gather(ref, [jnp.broadcast_to(gi, (_L,))])


def _sc_body(xr_h, yr_h, zr_h, xp_h, yp_h, zp_h, out_h,
             vxr, vyr, vzr, vsq, vxp, vyp, vzp, vd2, vacc,
             *, nbatch, rows_per_w):
    tecs_per_b = _NW // nbatch
    cid = lax.axis_index("c")
    sid = lax.axis_index("s")
    wid = sid * 2 + cid
    bat = wid // tecs_per_b
    row0 = (wid % tecs_per_b) * rows_per_w

    pltpu.sync_copy(xr_h.at[bat], vxr)
    pltpu.sync_copy(yr_h.at[bat], vyr)
    pltpu.sync_copy(zr_h.at[bat], vzr)
    pltpu.sync_copy(xp_h.at[bat], vxp)
    pltpu.sync_copy(yp_h.at[bat], vyp)
    pltpu.sync_copy(zp_h.at[bat], vzp)

    @pl.loop(0, _N, step=_L)
    def _(o):
        x = vxr[pl.ds(o, _L)]
        y = vyr[pl.ds(o, _L)]
        z = vzr[pl.ds(o, _L)]
        vsq[pl.ds(o, _L)] = x * x + y * y + z * z

    inf_v = jnp.full((_L,), jnp.inf, jnp.float32)
    iot = lax.iota(jnp.int32, _L)
    lane0 = iot == 0
    big = jnp.int32(_N)
    bigv = jnp.full((_L,), _N, jnp.int32)

    def block_min(base):
        bm = inf_v
        for u in range(8):
            bm = jnp.minimum(bm, vd2[pl.ds(base + u * _L, _L)])
        return bm

    def row_body(r, acc):
        gi = row0 + r
        giv = jnp.broadcast_to(gi, (_L,))
        xiv = plsc.load_gather(vxr, [giv])
        yiv = plsc.load_gather(vyr, [giv])
        ziv = plsc.load_gather(vzr, [giv])
        siv = xiv * xiv + yiv * yiv + ziv * ziv
        axv = -2.0 * xiv
        ayv = -2.0 * yiv
        azv = -2.0 * ziv

        def blk_body(bi, bv):
            v0, v1 = bv
            bm = inf_v
            base = bi * _BLKW
            for u in range(8):
                o = base + u * _L
                t = vsq[pl.ds(o, _L)] + vxr[pl.ds(o, _L)] * axv
                t = t + vyr[pl.ds(o, _L)] * ayv
                t = t + vzr[pl.ds(o, _L)] * azv
                vd2[pl.ds(o, _L)] = t
                bm = jnp.minimum(bm, t)
            mv = jnp.broadcast_to(jnp.min(bm), (_L,))
            v0 = jnp.where(iot == bi, mv, v0)
            v1 = jnp.where(iot == bi - _L, mv, v1)
            return (v0, v1)

        v0, v1 = lax.fori_loop(0, _NBLK, blk_body, (inf_v, inf_v),
                               unroll=False)

        plsc.store_scatter(vd2, [giv], inf_v, mask=lane0)
        dblk = gi // _BLKW
        mv = jnp.broadcast_to(jnp.min(block_min(dblk * _BLKW)), (_L,))
        v0 = jnp.where(iot == dblk, mv, v0)
        v1 = jnp.where(iot == dblk - _L, mv, v1)

        def ext_body(t, c):
            v0, v1, idxv, drefv = c
            m = jnp.min(jnp.minimum(v0, v1))
            mv = jnp.broadcast_to(m, (_L,))
            cand = jnp.minimum(jnp.where(v0 == mv, iot, bigv),
                               jnp.where(v1 == mv, iot + _L, bigv))
            sblk = jnp.min(cand)
            base = sblk * _BLKW
            posv = bigv
            for u in range(8):
                dv = vd2[pl.ds(base + u * _L, _L)]
                posv = jnp.minimum(posv,
                                   jnp.where(dv == mv, iot + u * _L, bigv))
            j = base + jnp.min(posv)
            idxv = jnp.where(iot == t, j, idxv)
            drefv = jnp.where(iot == t, mv, drefv)
            plsc.store_scatter(vd2, [jnp.broadcast_to(j, (_L,))], inf_v,
                               mask=lane0)
            m2v = jnp.broadcast_to(jnp.min(block_min(base)), (_L,))
            v0 = jnp.where(iot == sblk, m2v, v0)
            v1 = jnp.where(iot == sblk - _L, m2v, v1)
            return (v0, v1, idxv, drefv)

        _, _, idxv, drefv = lax.fori_loop(
            0, _K, ext_body, (v0, v1, jnp.zeros((_L,), jnp.int32), inf_v),
            unroll=False)

        dref = drefv + siv
        gxp = plsc.load_gather(vxp, [idxv])
        gyp = plsc.load_gather(vyp, [idxv])
        gzp = plsc.load_gather(vzp, [idxv])
        px = gxp - plsc.load_gather(vxp, [giv])
        py = gyp - plsc.load_gather(vyp, [giv])
        pz = gzp - plsc.load_gather(vzp, [giv])
        dp = px * px + py * py + pz * pz
        q = dp / dref
        qi = lax.bitcast_convert_type(q, jnp.int32)
        s = lax.bitcast_convert_type(
            jnp.full((_L,), 0x1FBD1DF5, jnp.int32)
            + lax.shift_right_logical(qi, 1), jnp.float32)
        for _ in range(3):
            s = 0.5 * (s + q / s)
        return acc + jnp.maximum(s - 1.0, 0.0)

    acc = lax.fori_loop(0, rows_per_w, row_body,
                        jnp.zeros((_L,), jnp.float32), unroll=False)
    vacc[...] = acc
    pltpu.sync_copy(vacc, out_h.at[wid])


def kernel(points_ref, points):
    nbatch, n, _ = points_ref.shape
    rows_per_w = nbatch * n // _NW
    mesh = plsc.VectorSubcoreMesh(core_axis_name="c", subcore_axis_name="s")
    body = functools.partial(_sc_body, nbatch=nbatch, rows_per_w=rows_per_w)
    run = pl.kernel(
        body,
        out_type=jax.ShapeDtypeStruct((_NW, _L), jnp.float32),
        mesh=mesh,
        compiler_params=pltpu.CompilerParams(needs_layout_passes=False),
        scratch_types=[
            pltpu.VMEM((n,), jnp.float32),
            pltpu.VMEM((n,), jnp.float32),
            pltpu.VMEM((n,), jnp.float32),
            pltpu.VMEM((n,), jnp.float32),
            pltpu.VMEM((n,), jnp.float32),
            pltpu.VMEM((n,), jnp.float32),
            pltpu.VMEM((n,), jnp.float32),
            pltpu.VMEM((n,), jnp.float32),
            pltpu.VMEM((_L,), jnp.float32),
        ],
    )
    pr = jnp.transpose(points_ref, (0, 2, 1))
    pp = jnp.transpose(points, (0, 2, 1))
    out = run(pr[:, 0], pr[:, 1], pr[:, 2], pp[:, 0], pp[:, 1], pp[:, 2])
    return jnp.sum(out) / jnp.float32(nbatch * n * _K)
